# Initial kernel scaffold; baseline (speedup 1.0000x reference)
import functools

import jax
import jax.numpy as jnp
from jax import lax
from jax.experimental import pallas as pl
from jax.experimental.pallas import tpu as pltpu
from jax.experimental.pallas import tpu_sc as plsc

VOCAB = 100000
DIM = 1024
MAX_SEQ = 8192
B = 4
S = 4096

NC = 2
NS = 16
LANES = 16
NW = NC * NS
NTOK = B * S
ROWS_PER_W = NTOK // NW
CHUNK = 16
NCH = ROWS_PER_W // CHUNK
NBUF = 2


def _body(tok_idx, pos_idx, word_tbl, pos_tbl, out,
          idx_t, idx_p, wbufs, pbufs, semw, semp, sems):
    wid = lax.axis_index("s") * NC + lax.axis_index("c")
    base = wid * ROWS_PER_W

    pltpu.sync_copy(tok_idx.at[wid], idx_t)
    pltpu.sync_copy(pos_idx.at[wid], idx_p)

    for bslot in range(NBUF):
        pltpu.async_copy(word_tbl.at[idx_t.at[bslot]], wbufs[bslot], semw[bslot])
        pltpu.async_copy(pos_tbl.at[idx_p.at[bslot]], pbufs[bslot], semp[bslot])

    def outer(g, carry):
        for bslot in range(NBUF):
            c = g * NBUF + bslot
            wb = wbufs[bslot]
            pb = pbufs[bslot]

            pltpu.make_async_copy(word_tbl.at[idx_t.at[c]], wb, semw[bslot]).wait()
            pltpu.make_async_copy(pos_tbl.at[idx_p.at[c]], pb, semp[bslot]).wait()

            @pl.when(c >= NBUF)
            def _():
                pltpu.make_async_copy(
                    wb, out.at[pl.ds(base, CHUNK)], sems[bslot]).wait()

            def add_row(r, _):
                for j in range(DIM // LANES):
                    sl = pl.ds(j * LANES, LANES)
                    wb[r, sl] = wb[r, sl] + pb[r, sl]
                return 0
            lax.fori_loop(0, CHUNK, add_row, 0)

            cn = c + NBUF

            @pl.when(cn < NCH)
            def _():
                pltpu.async_copy(word_tbl.at[idx_t.at[cn]], wb, semw[bslot])
                pltpu.async_copy(pos_tbl.at[idx_p.at[cn]], pb, semp[bslot])

            pltpu.async_copy(
                wb, out.at[pl.ds(base + c * CHUNK, CHUNK)], sems[bslot])
        return carry

    lax.fori_loop(0, NCH // NBUF, outer, 0)

    for bslot in range(NBUF):
        c = NCH - NBUF + bslot
        pltpu.make_async_copy(
            wbufs[bslot], out.at[pl.ds(base + c * CHUNK, CHUNK)],
            sems[bslot]).wait()


@jax.jit
def _run(tok_idx, pos_idx, word_table, pos_table):
    mesh = plsc.VectorSubcoreMesh(core_axis_name="c", subcore_axis_name="s")
    kfn = pl.kernel(
        _body,
        out_type=jax.ShapeDtypeStruct((NTOK, DIM), jnp.float32),
        mesh=mesh,
        scratch_types=[
            pltpu.VMEM((NCH, CHUNK), jnp.int32),
            pltpu.VMEM((NCH, CHUNK), jnp.int32),
            [pltpu.VMEM((CHUNK, DIM), jnp.float32) for _ in range(NBUF)],
            [pltpu.VMEM((CHUNK, DIM), jnp.float32) for _ in range(NBUF)],
            [pltpu.SemaphoreType.DMA for _ in range(NBUF)],
            [pltpu.SemaphoreType.DMA for _ in range(NBUF)],
            [pltpu.SemaphoreType.DMA for _ in range(NBUF)],
        ],
    )
    return kfn(tok_idx, pos_idx, word_table, pos_table)


def kernel(tokens, position_ids, word_table, pos_table):
    tok_idx = jnp.transpose(tokens, (1, 0)).astype(jnp.int32).reshape(
        NW, NCH, CHUNK)
    pos_idx = jnp.transpose(position_ids, (1, 0)).astype(jnp.int32).reshape(
        NW, NCH, CHUNK)
    flat = _run(tok_idx, pos_idx, word_table, pos_table)
    return flat.reshape(S, B, DIM)


# speedup vs baseline: 1.7722x; 1.7722x over previous
import jax
import jax.numpy as jnp
from jax import lax
from jax.experimental import pallas as pl
from jax.experimental.pallas import tpu as pltpu
from jax.experimental.pallas import tpu_sc as plsc

VOCAB = 100000
DIM = 1024
MAX_SEQ = 8192
B = 4
S = 4096

NC = 2
NS = 16
LANES = 16
NW = NC * NS
NTOK = B * S
ROWS_PER_W = NTOK // NW
CHUNK = 16
NCH = ROWS_PER_W // CHUNK
NBUF = 2


def _body(tok_idx, pos_idx, word_tbl, pos_tbl, out,
          idx_t, idx_p, wbufs, pbufs, sbufs, semw, semp, sems):
    wid = lax.axis_index("s") * NC + lax.axis_index("c")
    base = wid * ROWS_PER_W

    pltpu.sync_copy(tok_idx.at[wid], idx_t)
    pltpu.sync_copy(pos_idx.at[wid], idx_p)

    for bslot in range(NBUF):
        pltpu.async_copy(word_tbl.at[idx_t.at[bslot]], wbufs[bslot], semw[bslot])
        pltpu.async_copy(pos_tbl.at[idx_p.at[bslot]], pbufs[bslot], semp[bslot])

    def outer(g, carry):
        for bslot in range(NBUF):
            c = g * NBUF + bslot
            wb = wbufs[bslot]
            pb = pbufs[bslot]
            sb = sbufs[bslot]

            pltpu.make_async_copy(word_tbl.at[idx_t.at[c]], wb, semw[bslot]).wait()
            pltpu.make_async_copy(pos_tbl.at[idx_p.at[c]], pb, semp[bslot]).wait()

            @pl.when(c >= NBUF)
            def _():
                pltpu.make_async_copy(
                    sb, out.at[pl.ds(base, CHUNK)], sems[bslot]).wait()

            def add_row(r, _):
                for j in range(DIM // LANES):
                    sl = pl.ds(j * LANES, LANES)
                    sb[r, sl] = wb[r, sl] + pb[r, sl]
                return 0
            lax.fori_loop(0, CHUNK, add_row, 0)

            cn = c + NBUF

            @pl.when(cn < NCH)
            def _():
                pltpu.async_copy(word_tbl.at[idx_t.at[cn]], wb, semw[bslot])
                pltpu.async_copy(pos_tbl.at[idx_p.at[cn]], pb, semp[bslot])

            pltpu.async_copy(
                sb, out.at[pl.ds(base + c * CHUNK, CHUNK)], sems[bslot])
        return carry

    lax.fori_loop(0, NCH // NBUF, outer, 0)

    for bslot in range(NBUF):
        c = NCH - NBUF + bslot
        pltpu.make_async_copy(
            sbufs[bslot], out.at[pl.ds(base + c * CHUNK, CHUNK)],
            sems[bslot]).wait()


@jax.jit
def _run(tok_idx, pos_idx, word_table, pos_table):
    mesh = plsc.VectorSubcoreMesh(
        core_axis_name="c", subcore_axis_name="s",
        num_cores=NC, num_subcores=NS)
    kfn = pl.kernel(
        _body,
        out_type=jax.ShapeDtypeStruct((NTOK, DIM), jnp.float32),
        mesh=mesh,
        scratch_types=[
            pltpu.VMEM((NCH, CHUNK), jnp.int32),
            pltpu.VMEM((NCH, CHUNK), jnp.int32),
            [pltpu.VMEM((CHUNK, DIM), jnp.float32) for _ in range(NBUF)],
            [pltpu.VMEM((CHUNK, DIM), jnp.float32) for _ in range(NBUF)],
            [pltpu.VMEM((CHUNK, DIM), jnp.float32) for _ in range(NBUF)],
            [pltpu.SemaphoreType.DMA for _ in range(NBUF)],
            [pltpu.SemaphoreType.DMA for _ in range(NBUF)],
            [pltpu.SemaphoreType.DMA for _ in range(NBUF)],
        ],
    )
    return kfn(tok_idx, pos_idx, word_table, pos_table)


def kernel(tokens, position_ids, word_table, pos_table):
    tok_idx = jnp.transpose(tokens, (1, 0)).astype(jnp.int32).reshape(
        NW, NCH, CHUNK)
    pos_idx = jnp.transpose(position_ids, (1, 0)).astype(jnp.int32).reshape(
        NW, NCH, CHUNK)
    flat = _run(tok_idx, pos_idx, word_table, pos_table)
    return flat.reshape(S, B, DIM)
